# baseline (speedup 1.0000x reference)
import jax
import jax.numpy as jnp
from jax import lax
from jax.experimental import pallas as pl
from jax.experimental.pallas import tpu as pltpu

N_NODES = 50000
F = 16
F2 = 256
M = 512
H = 512
C = 128
ROW_BLK = 2000
GRID = -(-N_NODES // ROW_BLK)


def _fused_kernel(mask_ref, xcov_ref, bn1g_ref, bn1b_ref, W1_ref, b1_ref,
                  bn2g_ref, bn2b_ref, W2_ref, b2_ref, gum_ref,
                  cov2_ref, corr_ref, mout_ref):
    k = pl.program_id(0)
    pad = GRID * ROW_BLK - N_NODES
    if pad:
        @pl.when(k == GRID - 1)
        def _():
            mask_ref[pl.ds(ROW_BLK - pad, pad), :] = jnp.zeros((pad, M),
                                                               jnp.float32)
            xcov_ref[pl.ds(ROW_BLK - pad, pad), :] = jnp.zeros((pad, F2),
                                                               jnp.float32)

    part = lax.dot_general(mask_ref[:], xcov_ref[:], (((0,), (0,)), ((), ())),
                           preferred_element_type=jnp.float32)

    @pl.when(k == 0)
    def _():
        cov2_ref[:] = part

    @pl.when(k != 0)
    def _():
        cov2_ref[:] += part

    @pl.when(k == GRID - 1)
    def _():
        xc2 = cov2_ref[:]

        r256 = lax.broadcasted_iota(jnp.int32, (F2, F), 0)
        c16 = lax.broadcasted_iota(jnp.int32, (F2, F), 1)
        sel_diag = (r256 == c16 * (F + 1)).astype(jnp.float32)
        diag = lax.dot_general(xc2, sel_diag, (((1,), (0,)), ((), ())),
                               preferred_element_type=jnp.float32,
                               precision=lax.Precision.HIGHEST)
        var = lax.rsqrt(diag)

        rj = lax.broadcasted_iota(jnp.int32, (F, F2), 0)
        ck = lax.broadcasted_iota(jnp.int32, (F, F2), 1)
        sel_mod = (ck % F == rj).astype(jnp.float32)
        sel_div = (ck // F == rj).astype(jnp.float32)
        tiled = lax.dot_general(var, sel_mod, (((1,), (0,)), ((), ())),
                                preferred_element_type=jnp.float32,
                                precision=lax.Precision.HIGHEST)
        inter = lax.dot_general(var, sel_div, (((1,), (0,)), ((), ())),
                                preferred_element_type=jnp.float32,
                                precision=lax.Precision.HIGHEST)
        corr = tiled * inter * xc2
        x_corr = lax.dot_general(corr, sel_mod, (((1,), (1,)), ((), ())),
                                 preferred_element_type=jnp.float32,
                                 precision=lax.Precision.HIGHEST)
        corr_ref[:] = x_corr

        m1 = jnp.mean(x_corr, axis=0, keepdims=True)
        d1 = x_corr - m1
        v1 = jnp.mean(d1 * d1, axis=0, keepdims=True)
        xn = d1 / jnp.sqrt(v1 + 1e-5) * bn1g_ref[:] + bn1b_ref[:]
        h = lax.dot_general(xn.astype(jnp.bfloat16),
                            W1_ref[:].astype(jnp.bfloat16),
                            (((1,), (1,)), ((), ())),
                            preferred_element_type=jnp.float32) + b1_ref[:]
        h = jnp.maximum(h, 0.0)

        m2 = jnp.mean(h, axis=0, keepdims=True)
        d2 = h - m2
        v2 = jnp.mean(d2 * d2, axis=0, keepdims=True)
        hn = d2 / jnp.sqrt(v2 + 1e-5) * bn2g_ref[:] + bn2b_ref[:]
        logits = lax.dot_general(hn.astype(jnp.bfloat16),
                                 W2_ref[:].astype(jnp.bfloat16),
                                 (((1,), (1,)), ((), ())),
                                 preferred_element_type=jnp.float32) + b2_ref[:]

        z = logits + gum_ref[:]
        zmax = jnp.max(z, axis=1, keepdims=True)
        e = jnp.exp(z - zmax)
        y_soft = e / jnp.sum(e, axis=1, keepdims=True)
        ymax = jnp.max(y_soft, axis=1, keepdims=True)
        col = lax.broadcasted_iota(jnp.int32, (M, C), 1)
        cand = jnp.where(y_soft == ymax, col, C)
        idx = jnp.min(cand, axis=1, keepdims=True)
        y_hard = (col == idx).astype(jnp.float32)
        mout_ref[:] = (y_hard - y_soft) + y_soft


@jax.jit
def kernel(x_cov, mask, bn1_g, bn1_b, W1, b1, bn2_g, bn2_b, W2, b2,
           gumbel_noise):
    full = lambda shape: pl.BlockSpec(shape, lambda k: (0, 0))
    outs = pl.pallas_call(
        _fused_kernel,
        grid=(GRID,),
        in_specs=[
            pl.BlockSpec((ROW_BLK, M), lambda k: (k, 0)),
            pl.BlockSpec((ROW_BLK, F2), lambda k: (k, 0)),
            full((1, F)), full((1, F)),
            full((H, F)), full((1, H)),
            full((1, H)), full((1, H)),
            full((C, H)), full((1, C)),
            full((M, C)),
        ],
        out_specs=[full((M, F2)), full((M, F)), full((M, C))],
        out_shape=[
            jax.ShapeDtypeStruct((M, F2), jnp.float32),
            jax.ShapeDtypeStruct((M, F), jnp.float32),
            jax.ShapeDtypeStruct((M, C), jnp.float32),
        ],
        compiler_params=pltpu.CompilerParams(
            vmem_limit_bytes=100 * 1024 * 1024),
    )(mask, x_cov,
      bn1_g.reshape(1, F), bn1_b.reshape(1, F),
      W1, b1.reshape(1, H),
      bn2_g.reshape(1, H), bn2_b.reshape(1, H),
      W2, b2.reshape(1, C),
      gumbel_noise)
    return (outs[0], outs[1], outs[2])


# speedup vs baseline: 1.0310x; 1.0310x over previous
import jax
import jax.numpy as jnp
from jax import lax
from jax.experimental import pallas as pl
from jax.experimental.pallas import tpu as pltpu

N_NODES = 50000
F = 16
F2 = 256
M = 512
H = 512
C = 128
ROW_BLK = 7144
GRID = -(-N_NODES // ROW_BLK)


def _fused_kernel(mask_ref, xcov_ref, bn1g_ref, bn1b_ref, W1_ref, b1_ref,
                  bn2g_ref, bn2b_ref, W2_ref, b2_ref, gum_ref,
                  cov2_ref, corr_ref, mout_ref):
    k = pl.program_id(0)
    pad = GRID * ROW_BLK - N_NODES
    if pad:
        @pl.when(k == GRID - 1)
        def _():
            mask_ref[pl.ds(ROW_BLK - pad, pad), :] = jnp.zeros((pad, M),
                                                               jnp.float32)
            xcov_ref[pl.ds(ROW_BLK - pad, pad), :] = jnp.zeros((pad, F2),
                                                               jnp.float32)

    part = lax.dot_general(mask_ref[:], xcov_ref[:], (((0,), (0,)), ((), ())),
                           preferred_element_type=jnp.float32)

    @pl.when(k == 0)
    def _():
        cov2_ref[:] = part

    @pl.when(k != 0)
    def _():
        cov2_ref[:] += part

    @pl.when(k == GRID - 1)
    def _():
        xc2 = cov2_ref[:]

        r256 = lax.broadcasted_iota(jnp.int32, (F2, F), 0)
        c16 = lax.broadcasted_iota(jnp.int32, (F2, F), 1)
        sel_diag = (r256 == c16 * (F + 1)).astype(jnp.float32)
        diag = lax.dot_general(xc2, sel_diag, (((1,), (0,)), ((), ())),
                               preferred_element_type=jnp.float32,
                               precision=lax.Precision.HIGHEST)
        var = lax.rsqrt(diag)

        rj = lax.broadcasted_iota(jnp.int32, (F, F2), 0)
        ck = lax.broadcasted_iota(jnp.int32, (F, F2), 1)
        sel_mod = (ck % F == rj).astype(jnp.float32)
        sel_div = (ck // F == rj).astype(jnp.float32)
        tiled = lax.dot_general(var, sel_mod, (((1,), (0,)), ((), ())),
                                preferred_element_type=jnp.float32,
                                precision=lax.Precision.HIGHEST)
        inter = lax.dot_general(var, sel_div, (((1,), (0,)), ((), ())),
                                preferred_element_type=jnp.float32,
                                precision=lax.Precision.HIGHEST)
        corr = tiled * inter * xc2
        x_corr = lax.dot_general(corr, sel_mod, (((1,), (1,)), ((), ())),
                                 preferred_element_type=jnp.float32,
                                 precision=lax.Precision.HIGHEST)
        corr_ref[:] = x_corr

        m1 = jnp.mean(x_corr, axis=0, keepdims=True)
        d1 = x_corr - m1
        v1 = jnp.mean(d1 * d1, axis=0, keepdims=True)
        xn = d1 / jnp.sqrt(v1 + 1e-5) * bn1g_ref[:] + bn1b_ref[:]
        h = lax.dot_general(xn.astype(jnp.bfloat16),
                            W1_ref[:].astype(jnp.bfloat16),
                            (((1,), (1,)), ((), ())),
                            preferred_element_type=jnp.float32) + b1_ref[:]
        h = jnp.maximum(h, 0.0)

        m2 = jnp.mean(h, axis=0, keepdims=True)
        d2 = h - m2
        v2 = jnp.mean(d2 * d2, axis=0, keepdims=True)
        hn = d2 / jnp.sqrt(v2 + 1e-5) * bn2g_ref[:] + bn2b_ref[:]
        logits = lax.dot_general(hn.astype(jnp.bfloat16),
                                 W2_ref[:].astype(jnp.bfloat16),
                                 (((1,), (1,)), ((), ())),
                                 preferred_element_type=jnp.float32) + b2_ref[:]

        z = logits + gum_ref[:]
        zmax = jnp.max(z, axis=1, keepdims=True)
        e = jnp.exp(z - zmax)
        y_soft = e / jnp.sum(e, axis=1, keepdims=True)
        ymax = jnp.max(y_soft, axis=1, keepdims=True)
        col = lax.broadcasted_iota(jnp.int32, (M, C), 1)
        cand = jnp.where(y_soft == ymax, col, C)
        idx = jnp.min(cand, axis=1, keepdims=True)
        y_hard = (col == idx).astype(jnp.float32)
        mout_ref[:] = (y_hard - y_soft) + y_soft


@jax.jit
def kernel(x_cov, mask, bn1_g, bn1_b, W1, b1, bn2_g, bn2_b, W2, b2,
           gumbel_noise):
    full = lambda shape: pl.BlockSpec(shape, lambda k: (0, 0))
    outs = pl.pallas_call(
        _fused_kernel,
        grid=(GRID,),
        in_specs=[
            pl.BlockSpec((ROW_BLK, M), lambda k: (k, 0)),
            pl.BlockSpec((ROW_BLK, F2), lambda k: (k, 0)),
            full((1, F)), full((1, F)),
            full((H, F)), full((1, H)),
            full((1, H)), full((1, H)),
            full((C, H)), full((1, C)),
            full((M, C)),
        ],
        out_specs=[full((M, F2)), full((M, F)), full((M, C))],
        out_shape=[
            jax.ShapeDtypeStruct((M, F2), jnp.float32),
            jax.ShapeDtypeStruct((M, F), jnp.float32),
            jax.ShapeDtypeStruct((M, C), jnp.float32),
        ],
        compiler_params=pltpu.CompilerParams(
            vmem_limit_bytes=100 * 1024 * 1024),
    )(mask, x_cov,
      bn1_g.reshape(1, F), bn1_b.reshape(1, F),
      W1, b1.reshape(1, H),
      bn2_g.reshape(1, H), bn2_b.reshape(1, H),
      W2, b2.reshape(1, C),
      gumbel_noise)
    return (outs[0], outs[1], outs[2])
